# Initial kernel scaffold; baseline (speedup 1.0000x reference)
import jax
import jax.numpy as jnp
from jax.experimental import pallas as pl


def kernel(x1, edge_index1, x2, edge_index2):
    raise NotImplementedError("write your pallas kernel here")


# speedup vs baseline: 745.1533x; 745.1533x over previous
import functools

import jax
import jax.numpy as jnp
from jax import lax
from jax.experimental import pallas as pl
from jax.experimental.pallas import tpu as pltpu
from jax.experimental.pallas import tpu_sc as plsc

N = 100000
E = 1600000
P1 = 1000003
P2 = 0x85EBCA6B
C1 = 0x9E3779B9
NUM_LAYERS = 3

NTILES = 16
EPT = E // NTILES
CH = 2000
WB = N // 4
ZB = ((WB + 15) // 16) * 16


def _segsum_body(table, srcg, dstl, out, acc, src_buf, dst_buf, val_buf,
                 zbuf, gsem):
    i32 = jnp.int32
    c = lax.axis_index("c").astype(jnp.int32)
    s = lax.axis_index("s").astype(jnp.int32)
    zero16 = jnp.zeros((16,), jnp.int32)

    def zfill(i, carry):
        zbuf[pl.ds(i * i32(16), 16)] = zero16
        return carry

    lax.fori_loop(i32(0), i32(ZB // 16), zfill, i32(0))

    @pl.when(s < 4)
    def _():
        pltpu.sync_copy(zbuf.at[pl.ds(0, WB)], acc.at[pl.ds(s * i32(WB), WB)])

    plsc.subcore_barrier()

    ebase = c * i32(E) + s * i32(EPT)

    def chunk(k, carry):
        b = ebase + k * i32(CH)
        pltpu.sync_copy(srcg.at[pl.ds(b, CH)], src_buf)
        pltpu.sync_copy(dstl.at[pl.ds(b, CH)], dst_buf)
        pltpu.async_copy(table.at[src_buf], val_buf, gsem).wait()
        pltpu.sync_copy(val_buf, acc.at[dst_buf], add=True)
        return carry

    lax.fori_loop(i32(0), i32(EPT // CH), chunk, i32(0))

    plsc.subcore_barrier()

    @pl.when(s < 4)
    def _():
        pltpu.sync_copy(acc.at[pl.ds(s * i32(WB), WB)], zbuf.at[pl.ds(0, WB)])
        pltpu.sync_copy(zbuf.at[pl.ds(0, WB)],
                        out.at[pl.ds(c * i32(N) + s * i32(WB), WB)])


_segsum = functools.partial(
    pl.kernel,
    out_type=jax.ShapeDtypeStruct((2 * N,), jnp.int32),
    mesh=plsc.VectorSubcoreMesh(core_axis_name="c", subcore_axis_name="s",
                                num_cores=2, num_subcores=16),
    scratch_types=[
        pltpu.VMEM_SHARED((N,), jnp.int32),
        pltpu.VMEM((CH,), jnp.int32),
        pltpu.VMEM((CH,), jnp.int32),
        pltpu.VMEM((CH,), jnp.int32),
        pltpu.VMEM((ZB,), jnp.int32),
        pltpu.SemaphoreType.DMA,
    ],
)(_segsum_body)

K = 2 * N
KP = 200704
PT = KP // 16
NV = PT // 16
NB = 512
CNT_BASE = 1


def _digit(lo, hi, p):
    i32 = jnp.int32
    if p == 0:
        return lax.bitwise_and(lo, i32(0x1FF))
    if p == 1:
        return lax.bitwise_and(lax.shift_right_logical(lo, i32(9)), i32(0x1FF))
    if p == 2:
        return lax.bitwise_and(lax.shift_right_logical(lo, i32(18)), i32(0x1FF))
    if p == 3:
        lo_part = lax.shift_right_logical(lo, i32(27))
        hi_part = lax.shift_left(lax.bitwise_and(hi, i32(0xF)), i32(5))
        return lax.bitwise_or(lo_part, hi_part)
    if p == 4:
        return lax.bitwise_and(lax.shift_right_logical(hi, i32(4)), i32(0x1FF))
    return lax.bitwise_and(lax.shift_right_logical(hi, i32(13)), i32(0x1FF))


def _rank_body(klo, khi, table, out_d, a_pay, b_pay, a_lo, a_hi, b_lo, b_hi,
               grid_sp, misc_sp, ddiff_sp, o_lo, o_hi, o_pay, v_lo, v_hi,
               v_pay, v_dig, v_pos, v_hist, v_cur, v_big, gsem):
    i32 = jnp.int32
    c = lax.axis_index("c").astype(jnp.int32)
    s = lax.axis_index("s").astype(jnp.int32)
    lanes = lax.iota(jnp.int32, 16)
    zero16 = jnp.zeros((16,), jnp.int32)
    tbase = s * i32(PT)
    cb_vec, _ = plsc.scan_count(zero16)
    cb = cb_vec[0]

    @pl.when(c == 0)
    def _():
        def zf(i, carry):
            v_pos[pl.ds(i * i32(16), 16)] = zero16
            return carry
        lax.fori_loop(i32(0), i32(NV), zf, i32(0))
        pltpu.sync_copy(v_pos.at[pl.ds(0, PT)], ddiff_sp.at[pl.ds(tbase, PT)])

    for p in range(6):
        @pl.when(c == 0)
        def _(p=p):
            if p == 0:
                pltpu.sync_copy(klo.at[pl.ds(tbase, PT)], v_lo.at[pl.ds(0, PT)])
                pltpu.sync_copy(khi.at[pl.ds(tbase, PT)], v_hi.at[pl.ds(0, PT)])
            else:
                src_lo, src_hi, src_pay = ((a_lo, a_hi, a_pay) if p % 2 == 1
                                           else (b_lo, b_hi, b_pay))
                pltpu.sync_copy(src_lo.at[pl.ds(tbase, PT)], v_lo.at[pl.ds(0, PT)])
                pltpu.sync_copy(src_hi.at[pl.ds(tbase, PT)], v_hi.at[pl.ds(0, PT)])
                pltpu.sync_copy(src_pay.at[pl.ds(tbase, PT)], v_pay.at[pl.ds(0, PT)])

            def hzero(i, carry):
                v_hist[pl.ds(i * i32(16), 16)] = zero16
                return carry
            lax.fori_loop(i32(0), i32(NB // 16), hzero, i32(0))

            def hloop(i, carry):
                o = i * i32(16)
                dig = _digit(v_lo[pl.ds(o, 16)], v_hi[pl.ds(o, 16)], p)
                v_dig[pl.ds(o, 16)] = dig
                cnt, last = plsc.scan_count(dig)
                plsc.addupdate_scatter(v_hist, [dig],
                                       cnt - cb + i32(1), mask=last)
                return carry
            lax.fori_loop(i32(0), i32(NV), hloop, i32(0))
            pltpu.sync_copy(v_hist.at[pl.ds(0, NB)],
                            grid_sp.at[pl.ds(s * i32(NB), NB)])

        plsc.subcore_barrier()

        @pl.when(c == 0)
        def _(p=p):
            pltpu.sync_copy(grid_sp.at[pl.ds(0, 16 * NB)],
                            v_big.at[pl.ds(0, 16 * NB)])

            def sloop(j, carry):
                o = j * i32(16)
                tot = zero16
                pre = zero16
                for t in range(16):
                    g = v_big[pl.ds(i32(t * NB) + o, 16)]
                    tot = tot + g
                    pre = pre + jnp.where(i32(t) < s, g, i32(0))
                incl = plsc.cumsum(tot)
                excl = incl - tot
                carry_vec = jnp.full((16,), carry, jnp.int32)
                v_cur[pl.ds(o, 16)] = carry_vec + excl + pre
                return carry + incl[15]
            lax.fori_loop(i32(0), i32(NB // 16), sloop, i32(0))

            def ploop(i, carry):
                o = i * i32(16)
                dig = v_dig[pl.ds(o, 16)]
                base = plsc.load_gather(v_cur, [dig])
                cnt, last = plsc.scan_count(dig)
                cnt0 = cnt - cb
                v_pos[pl.ds(o, 16)] = base + cnt0
                plsc.addupdate_scatter(v_cur, [dig], cnt0 + i32(1), mask=last)
                if p == 0:
                    v_pay[pl.ds(o, 16)] = tbase + o + lanes
                return carry
            lax.fori_loop(i32(0), i32(NV), ploop, i32(0))

            pltpu.sync_copy(v_lo.at[pl.ds(0, PT)], o_lo.at[v_pos])
            pltpu.sync_copy(v_hi.at[pl.ds(0, PT)], o_hi.at[v_pos])
            pltpu.sync_copy(v_pay.at[pl.ds(0, PT)], o_pay.at[v_pos])

        plsc.subcore_barrier()

        @pl.when(c == 0)
        def _(p=p):
            dst_lo, dst_hi, dst_pay = ((a_lo, a_hi, a_pay) if p % 2 == 0
                                       else (b_lo, b_hi, b_pay))
            pltpu.sync_copy(o_lo.at[pl.ds(tbase, PT)], v_lo.at[pl.ds(0, PT)])
            pltpu.sync_copy(v_lo.at[pl.ds(0, PT)], dst_lo.at[pl.ds(tbase, PT)])
            pltpu.sync_copy(o_hi.at[pl.ds(tbase, PT)], v_hi.at[pl.ds(0, PT)])
            pltpu.sync_copy(v_hi.at[pl.ds(0, PT)], dst_hi.at[pl.ds(tbase, PT)])
            pltpu.sync_copy(o_pay.at[pl.ds(tbase, PT)], v_pay.at[pl.ds(0, PT)])
            pltpu.sync_copy(v_pay.at[pl.ds(0, PT)],
                            dst_pay.at[pl.ds(tbase, PT)])

        plsc.subcore_barrier()

    @pl.when(c == 0)
    def _():
        pltpu.sync_copy(b_lo.at[pl.ds(tbase, PT)], v_lo.at[pl.ds(16, PT)])
        pltpu.sync_copy(b_hi.at[pl.ds(tbase, PT)], v_hi.at[pl.ds(16, PT)])
        pltpu.sync_copy(b_pay.at[pl.ds(tbase, PT)], v_pay.at[pl.ds(0, PT)])

        @pl.when(s == 0)
        def _():
            v_lo[pl.ds(0, 16)] = jnp.full((16,), -1, jnp.int32)
            v_hi[pl.ds(0, 16)] = jnp.full((16,), -1, jnp.int32)

        @pl.when(s > 0)
        def _():
            pltpu.sync_copy(b_lo.at[pl.ds(tbase - i32(16), 16)],
                            v_lo.at[pl.ds(0, 16)])
            pltpu.sync_copy(b_hi.at[pl.ds(tbase - i32(16), 16)],
                            v_hi.at[pl.ds(0, 16)])

        def lloop(i, carry):
            o = i * i32(16)
            lo = v_lo[pl.ds(o + i32(16), 16)]
            hi = v_hi[pl.ds(o + i32(16), 16)]
            plo = v_lo[pl.ds(o + i32(15), 16)]
            phi = v_hi[pl.ds(o + i32(15), 16)]
            nr = jnp.where((lo != plo) | (hi != phi), i32(1), i32(0))
            pay = v_pay[pl.ds(o, 16)]
            zv = jnp.where(pay >= i32(N), i32(-1), i32(1))
            rinc = plsc.cumsum(nr)
            v_dig[pl.ds(o, 16)] = carry + rinc
            v_pos[pl.ds(o, 16)] = zv
            return carry + rinc[15]
        rank_tot = lax.fori_loop(i32(0), i32(NV), lloop, i32(0))

        v_big[pl.ds(0, 16)] = jnp.where(lanes == 0, rank_tot, i32(0))
        pltpu.sync_copy(v_big.at[pl.ds(0, 16)],
                        misc_sp.at[pl.ds(s * i32(16), 16)])

    plsc.subcore_barrier()

    @pl.when(c == 0)
    def _():
        pltpu.sync_copy(misc_sp.at[pl.ds(0, 256)], v_big.at[pl.ds(0, 256)])
        rank_off = i32(0)
        for t in range(16):
            rank_off = rank_off + jnp.where(
                i32(t) < s, v_big[pl.ds(i32(t * 16), 16)][0], i32(0))

        def wloop(i, carry):
            o = i * i32(16)
            rank = v_dig[pl.ds(o, 16)] + rank_off - i32(1)
            v_dig[pl.ds(o, 16)] = rank
            v_hi[pl.ds(o, 16)] = rank * i32(128) + i32(1)
            return carry
        lax.fori_loop(i32(0), i32(NV), wloop, i32(0))
        pltpu.sync_copy(v_hi.at[pl.ds(0, PT)], o_lo.at[v_pay])
        pltpu.sync_copy(v_pos.at[pl.ds(0, PT)], ddiff_sp.at[v_dig], add=True)

    plsc.subcore_barrier()

    @pl.when(c == 0)
    def _():
        pltpu.sync_copy(o_lo.at[pl.ds(tbase, PT)], v_hi.at[pl.ds(0, PT)])
        pltpu.sync_copy(v_hi.at[pl.ds(0, PT)], table.at[pl.ds(tbase, PT)])
        pltpu.sync_copy(ddiff_sp.at[pl.ds(tbase, PT)], v_lo.at[pl.ds(0, PT)])

        def dloop(i, acc):
            o = i * i32(16)
            return acc + jnp.abs(v_lo[pl.ds(o, 16)])
        acc = lax.fori_loop(i32(0), i32(NV), dloop, zero16)
        dsum = lax.reduce_sum_p.bind(acc, axes=(0,))
        v_big[pl.ds(0, 16)] = jnp.where(lanes == 0, dsum, i32(0))
        pltpu.sync_copy(v_big.at[pl.ds(0, 16)],
                        out_d.at[pl.ds(s * i32(16), 16)])


_rank = functools.partial(
    pl.kernel,
    out_type=(jax.ShapeDtypeStruct((KP,), jnp.int32),
              jax.ShapeDtypeStruct((256,), jnp.int32),
              jax.ShapeDtypeStruct((KP,), jnp.int32),
              jax.ShapeDtypeStruct((KP,), jnp.int32),
              jax.ShapeDtypeStruct((KP,), jnp.int32),
              jax.ShapeDtypeStruct((KP,), jnp.int32),
              jax.ShapeDtypeStruct((KP,), jnp.int32),
              jax.ShapeDtypeStruct((KP,), jnp.int32)),
    mesh=plsc.VectorSubcoreMesh(core_axis_name="c", subcore_axis_name="s",
                                num_cores=2, num_subcores=16),
    compiler_params=pltpu.CompilerParams(needs_layout_passes=False),
    scratch_types=[
        pltpu.VMEM_SHARED((16 * NB,), jnp.int32),
        pltpu.VMEM_SHARED((256,), jnp.int32),
        pltpu.VMEM_SHARED((KP,), jnp.int32),
        pltpu.VMEM_SHARED((KP,), jnp.int32),
        pltpu.VMEM_SHARED((KP,), jnp.int32),
        pltpu.VMEM_SHARED((KP,), jnp.int32),
        pltpu.VMEM((PT + 16,), jnp.int32),
        pltpu.VMEM((PT + 16,), jnp.int32),
        pltpu.VMEM((PT,), jnp.int32),
        pltpu.VMEM((PT,), jnp.int32),
        pltpu.VMEM((PT,), jnp.int32),
        pltpu.VMEM((NB,), jnp.int32),
        pltpu.VMEM((NB,), jnp.int32),
        pltpu.VMEM((16 * NB,), jnp.int32),
        pltpu.SemaphoreType.DMA,
    ],
)(_rank_body)

NPAD = KP - K


def kernel(x1, edge_index1, x2, edge_index2):
    src1 = edge_index1[0].astype(jnp.int32)
    dst1 = edge_index1[1].astype(jnp.int32)
    src2 = edge_index2[0].astype(jnp.int32)
    dst2 = edge_index2[1].astype(jnp.int32)
    srcg = jnp.concatenate([src1, src2 + N])
    dstl = jnp.concatenate([dst1, dst2])

    c1 = jnp.clip((x1 * 32).astype(jnp.int32), 0, 31)
    c2 = jnp.clip((x2 * 32).astype(jnp.int32), 0, 31)
    colors = jnp.concatenate([c1, c2])

    pad_lo = jnp.full((NPAD,), -1, jnp.int32)
    pad_hi = jnp.full((NPAD,), 0x7FFFFF, jnp.int32)

    d = jnp.float32(0.0)
    table = colors * 128 + 1
    for _ in range(NUM_LAYERS):
        packed = _segsum(table, srcg, dstl)
        S = (packed >> 7).astype(jnp.int64)
        D = (packed & 127).astype(jnp.int64)
        sig = (colors.astype(jnp.int64) * P2 + jnp.int64(P1) * S
               + jnp.int64(C1) * D)
        klo = jnp.concatenate([(sig & 0xFFFFFFFF).astype(jnp.int32)
                               .astype(jnp.int32), pad_lo])
        khi = jnp.concatenate([(sig >> 32).astype(jnp.int32), pad_hi])
        table_kp, dpart = _rank(klo, khi)[:2]
        table = table_kp[:K]
        colors = table >> 7
        d = d + (dpart.sum().astype(jnp.float32) - jnp.float32(NPAD))
    return jnp.float32(1.0) - d / jnp.float32(NUM_LAYERS * K)
